# baseline (speedup 1.0000x reference)
import functools

import jax
import jax.numpy as jnp
from jax import lax
from jax.experimental import pallas as pl
from jax.experimental.pallas import tpu as pltpu
from jax.experimental.pallas import tpu_sc as plsc

VOCAB = 100000
EMBED = 32
B = 16384
L = 50
N = B * L

NC = 2
NS = 16
NW = NC * NS
B_PER_W = N // NW
CHUNK = 128
K = 4
G = K * CHUNK
N_GROUPS = B_PER_W // G
ROWS_PER_W = B_PER_W // CHUNK
LAST = N_GROUPS - 1
NLANE = 16
SUB = EMBED // 8


def _sc_gather(idx2d, table):
    mesh = plsc.VectorSubcoreMesh(core_axis_name="c", subcore_axis_name="s")

    @functools.partial(
        pl.kernel,
        mesh=mesh,
        compiler_params=pltpu.CompilerParams(
            use_tc_tiling_on_sc=False, needs_layout_passes=False
        ),
        out_type=jax.ShapeDtypeStruct((SUB, N // CHUNK, 8, CHUNK), jnp.float32),
        scratch_types=[
            pltpu.VMEM((2 * K, CHUNK), jnp.int32),
            pltpu.VMEM((2, G, EMBED), jnp.float32),
            pltpu.VMEM((2, K, EMBED, CHUNK), jnp.float32),
            pltpu.SemaphoreType.DMA,
            pltpu.SemaphoreType.DMA,
        ],
    )
    def body(idx_hbm, table_hbm, out_hbm, idx_v, rows_v, stage_v, sem_g, sem_o):
        wid = lax.axis_index("s") * NC + lax.axis_index("c")
        row0 = wid * ROWS_PER_W

        def fire(buf, r):
            for j in range(K):
                pltpu.async_copy(
                    table_hbm.at[idx_v.at[buf * K + j]],
                    rows_v.at[buf].at[pl.ds(j * CHUNK, CHUNK)],
                    sem_g,
                )

        def out_descs(buf, r):
            return [
                pltpu.make_async_copy(
                    stage_v.at[buf].at[:, pl.ds(8 * s, 8), :],
                    out_hbm.at[s].at[pl.ds(r, K)],
                    sem_o,
                )
                for s in range(SUB)
            ]

        def transpose(buf):
            rows_b = rows_v.at[buf]
            stage_b = stage_v.at[buf]
            iota = jax.lax.iota(jnp.int32, NLANE)

            cols = [jnp.full((NLANE,), e, jnp.int32) for e in range(EMBED)]

            @plsc.parallel_loop(0, CHUNK // NLANE, unroll=4)
            def j_body(j0):
                base = j0 * NLANE
                for k in range(K):
                    row = iota + base + (k * CHUNK)
                    for e in range(EMBED):
                        v = plsc.load_gather(rows_b, [row, cols[e]])
                        stage_b[k, e, pl.ds(base, NLANE)] = v

        def step(g, cur):
            nxt = 1 - cur
            static = isinstance(g, int)
            r = row0 + g * K

            def when(cond_static, cond_traced, fn):
                if static:
                    if cond_static:
                        fn()
                else:
                    pl.when(cond_traced)(fn)

            when(g < LAST, g < LAST, lambda: pltpu.sync_copy(
                idx_hbm.at[pl.ds(r + K, K)],
                idx_v.at[pl.ds(nxt * K, K)],
            ))

            pltpu.make_async_copy(
                table_hbm.at[pl.ds(0, G)],
                rows_v.at[cur],
                sem_g,
            ).wait()

            when(g < LAST, g < LAST, lambda: fire(nxt, r + K))

            def retire():
                for d in out_descs(cur, row0):
                    d.wait()

            when(g >= 2, g >= 2, retire)

            transpose(cur)

            for d in out_descs(cur, r):
                d.start()

        pltpu.sync_copy(idx_hbm.at[pl.ds(row0, K)], idx_v.at[pl.ds(0, K)])
        fire(0, row0)
        step(0, 0)

        def pair(i, carry):
            g = 2 * i + 1
            step(g, 1)
            step(g + 1, 0)
            return carry

        lax.fori_loop(0, (N_GROUPS - 2) // 2, pair, 0)
        step(LAST, LAST % 2)

        for buf in ((LAST - 1) % 2, LAST % 2):
            for d in out_descs(buf, row0):
                d.wait()

    return body(idx2d, table)


def kernel(inputs, table):
    idx2d = inputs.astype(jnp.int32).reshape(N // CHUNK, CHUNK)
    out4d = _sc_gather(idx2d, table)
    return out4d.transpose(1, 3, 0, 2).reshape(N, EMBED)


# speedup vs baseline: 1.1916x; 1.1916x over previous
import functools

import jax
import jax.numpy as jnp
from jax import lax
from jax.experimental import pallas as pl
from jax.experimental.pallas import tpu as pltpu
from jax.experimental.pallas import tpu_sc as plsc

VOCAB = 100000
EMBED = 32
B = 16384
L = 50
N = B * L

NC = 2
NS = 16
NW = NC * NS
B_PER_W = N // NW
CHUNK = 128
K = 4
G = K * CHUNK
N_GROUPS = B_PER_W // G
ROWS_PER_W = B_PER_W // CHUNK
LAST = N_GROUPS - 1
NLANE = 16
SUB = EMBED // 8


def _sc_gather(idx2d, table):
    mesh = plsc.VectorSubcoreMesh(core_axis_name="c", subcore_axis_name="s")

    @functools.partial(
        pl.kernel,
        mesh=mesh,
        compiler_params=pltpu.CompilerParams(
            use_tc_tiling_on_sc=False, needs_layout_passes=False
        ),
        out_type=jax.ShapeDtypeStruct((SUB, N // CHUNK, 8, CHUNK), jnp.float32),
        scratch_types=[
            pltpu.VMEM((2 * K, CHUNK), jnp.int32),
            pltpu.VMEM((2, G, EMBED), jnp.float32),
            pltpu.VMEM((2, K, EMBED, CHUNK), jnp.float32),
            pltpu.SemaphoreType.DMA,
            pltpu.SemaphoreType.DMA,
        ],
    )
    def body(idx_hbm, table_hbm, out_hbm, idx_v, rows_v, stage_v, sem_g, sem_o):
        wid = lax.axis_index("s") * NC + lax.axis_index("c")
        row0 = wid * ROWS_PER_W

        def fire(buf, r):
            for j in range(K):
                pltpu.async_copy(
                    table_hbm.at[idx_v.at[buf * K + j]],
                    rows_v.at[buf].at[pl.ds(j * CHUNK, CHUNK)],
                    sem_g,
                )

        def out_descs(buf, r):
            return [
                pltpu.make_async_copy(
                    stage_v.at[buf].at[:, pl.ds(8 * s, 8), :],
                    out_hbm.at[s].at[pl.ds(r, K)],
                    sem_o,
                )
                for s in range(SUB)
            ]

        def transpose(buf):
            rows_b = rows_v.at[buf]
            stage_b = stage_v.at[buf]
            iota = jax.lax.iota(jnp.int32, NLANE)

            cols = [jnp.full((NLANE,), e, jnp.int32) for e in range(EMBED)]

            @plsc.parallel_loop(0, CHUNK // NLANE, unroll=2)
            def j_body(j0):
                base = j0 * NLANE
                for k in range(K):
                    row = iota + base + (k * CHUNK)
                    for e in range(EMBED):
                        v = plsc.load_gather(rows_b, [row, cols[e]])
                        stage_b[k, e, pl.ds(base, NLANE)] = v

        def step(g, cur):
            nxt = 1 - cur
            static = isinstance(g, int)
            r = row0 + g * K

            def when(cond_static, cond_traced, fn):
                if static:
                    if cond_static:
                        fn()
                else:
                    pl.when(cond_traced)(fn)

            when(g < LAST, g < LAST, lambda: pltpu.sync_copy(
                idx_hbm.at[pl.ds(r + K, K)],
                idx_v.at[pl.ds(nxt * K, K)],
            ))

            pltpu.make_async_copy(
                table_hbm.at[pl.ds(0, G)],
                rows_v.at[cur],
                sem_g,
            ).wait()

            when(g < LAST, g < LAST, lambda: fire(nxt, r + K))

            def retire():
                for d in out_descs(cur, row0):
                    d.wait()

            when(g >= 2, g >= 2, retire)

            transpose(cur)

            for d in out_descs(cur, r):
                d.start()

        pltpu.sync_copy(idx_hbm.at[pl.ds(row0, K)], idx_v.at[pl.ds(0, K)])
        fire(0, row0)
        step(0, 0)

        def pair(i, carry):
            g = 2 * i + 1
            step(g, 1)
            step(g + 1, 0)
            return carry

        lax.fori_loop(0, (N_GROUPS - 2) // 2, pair, 0)
        step(LAST, LAST % 2)

        for buf in ((LAST - 1) % 2, LAST % 2):
            for d in out_descs(buf, row0):
                d.wait()

    return body(idx2d, table)


def kernel(inputs, table):
    idx2d = inputs.astype(jnp.int32).reshape(N // CHUNK, CHUNK)
    out4d = _sc_gather(idx2d, table)
    return out4d.transpose(1, 3, 0, 2).reshape(N, EMBED)


# speedup vs baseline: 2.3599x; 1.9805x over previous
import functools

import jax
import jax.numpy as jnp
from jax import lax
from jax.experimental import pallas as pl
from jax.experimental.pallas import tpu as pltpu
from jax.experimental.pallas import tpu_sc as plsc

VOCAB = 100000
EMBED = 32
B = 16384
L = 50
N = B * L

NC = 2
NS = 16
NW = NC * NS
B_PER_W = N // NW
CHUNK = 128
K = 4
G = K * CHUNK
N_GROUPS = B_PER_W // G
ROWS_PER_W = B_PER_W // CHUNK
LAST = N_GROUPS - 1
NLANE = 16
SUB = EMBED // 8


def _sc_gather(idx2d, table):
    mesh = plsc.VectorSubcoreMesh(core_axis_name="c", subcore_axis_name="s")

    @functools.partial(
        pl.kernel,
        mesh=mesh,
        compiler_params=pltpu.CompilerParams(
            use_tc_tiling_on_sc=False, needs_layout_passes=False
        ),
        out_type=jax.ShapeDtypeStruct((SUB, N // CHUNK, 8, CHUNK), jnp.float32),
        scratch_types=[
            pltpu.VMEM((2 * K, CHUNK), jnp.int32),
            pltpu.VMEM((2, G, EMBED), jnp.float32),
            pltpu.VMEM((2, K, EMBED, CHUNK), jnp.float32),
            pltpu.SemaphoreType.DMA,
            pltpu.SemaphoreType.DMA,
        ],
    )
    def body(idx_hbm, table_hbm, out_hbm, idx_v, rows_v, stage_v, sem_g, sem_o):
        wid = lax.axis_index("s") * NC + lax.axis_index("c")
        row0 = wid * ROWS_PER_W

        def fire(buf, r):
            for j in range(K):
                pltpu.async_copy(
                    table_hbm.at[idx_v.at[buf * K + j]],
                    rows_v.at[buf].at[pl.ds(j * CHUNK, CHUNK)],
                    sem_g,
                )

        def out_descs(buf, r):
            return [
                pltpu.make_async_copy(
                    stage_v.at[buf].at[:, pl.ds(8 * s, 8), :],
                    out_hbm.at[s].at[pl.ds(r, K)],
                    sem_o,
                )
                for s in range(SUB)
            ]

        def transpose(buf):
            iota = jax.lax.iota(jnp.int32, NLANE)
            cols_mod = [(iota + e) % EMBED for e in range(EMBED)]
            rows_b = rows_v.at[buf]
            stage_b = stage_v.at[buf]

            @plsc.parallel_loop(0, G // NLANE, unroll=2)
            def t_body(t):
                rvec = iota + t * NLANE
                k = t // (CHUNK // NLANE)
                kvec = jnp.zeros((NLANE,), jnp.int32) + k
                jvec = rvec - k * CHUNK
                for e in range(EMBED):
                    v = plsc.load_gather(rows_b, [rvec, cols_mod[e]])
                    plsc.store_scatter(stage_b, [kvec, cols_mod[e], jvec], v)

        def step(g, cur):
            nxt = 1 - cur
            static = isinstance(g, int)
            r = row0 + g * K

            def when(cond_static, cond_traced, fn):
                if static:
                    if cond_static:
                        fn()
                else:
                    pl.when(cond_traced)(fn)

            when(g < LAST, g < LAST, lambda: pltpu.sync_copy(
                idx_hbm.at[pl.ds(r + K, K)],
                idx_v.at[pl.ds(nxt * K, K)],
            ))

            pltpu.make_async_copy(
                table_hbm.at[pl.ds(0, G)],
                rows_v.at[cur],
                sem_g,
            ).wait()

            when(g < LAST, g < LAST, lambda: fire(nxt, r + K))

            def retire():
                for d in out_descs(cur, row0):
                    d.wait()

            when(g >= 2, g >= 2, retire)

            transpose(cur)

            for d in out_descs(cur, r):
                d.start()

        pltpu.sync_copy(idx_hbm.at[pl.ds(row0, K)], idx_v.at[pl.ds(0, K)])
        fire(0, row0)
        step(0, 0)

        def pair(i, carry):
            g = 2 * i + 1
            step(g, 1)
            step(g + 1, 0)
            return carry

        lax.fori_loop(0, (N_GROUPS - 2) // 2, pair, 0)
        step(LAST, LAST % 2)

        for buf in ((LAST - 1) % 2, LAST % 2):
            for d in out_descs(buf, row0):
                d.wait()

    return body(idx2d, table)


def kernel(inputs, table):
    idx2d = inputs.astype(jnp.int32).reshape(N // CHUNK, CHUNK)
    out4d = _sc_gather(idx2d, table)
    return out4d.transpose(1, 3, 0, 2).reshape(N, EMBED)


# speedup vs baseline: 2.5377x; 1.0753x over previous
import functools

import jax
import jax.numpy as jnp
from jax import lax
from jax.experimental import pallas as pl
from jax.experimental.pallas import tpu as pltpu
from jax.experimental.pallas import tpu_sc as plsc

VOCAB = 100000
EMBED = 32
B = 16384
L = 50
N = B * L

NC = 2
NS = 16
NW = NC * NS
B_PER_W = N // NW
CHUNK = 128
K = 4
G = K * CHUNK
N_GROUPS = B_PER_W // G
ROWS_PER_W = B_PER_W // CHUNK
LAST = N_GROUPS - 1
NLANE = 16
SUB = EMBED // 8


def _sc_gather(idx2d, table):
    mesh = plsc.VectorSubcoreMesh(core_axis_name="c", subcore_axis_name="s")

    @functools.partial(
        pl.kernel,
        mesh=mesh,
        compiler_params=pltpu.CompilerParams(
            use_tc_tiling_on_sc=False, needs_layout_passes=False
        ),
        out_type=jax.ShapeDtypeStruct((SUB, N // CHUNK, 8, CHUNK), jnp.float32),
        scratch_types=[
            pltpu.VMEM((2 * K, CHUNK), jnp.int32),
            pltpu.VMEM((2, G, EMBED), jnp.float32),
            pltpu.VMEM((2, K, EMBED, CHUNK), jnp.float32),
            pltpu.SemaphoreType.DMA,
            pltpu.SemaphoreType.DMA,
        ],
    )
    def body(idx_hbm, table_hbm, out_hbm, idx_v, rows_v, stage_v, sem_g, sem_o):
        wid = lax.axis_index("s") * NC + lax.axis_index("c")
        row0 = wid * ROWS_PER_W

        def fire(buf, r):
            for j in range(K):
                pltpu.async_copy(
                    table_hbm.at[idx_v.at[buf * K + j]],
                    rows_v.at[buf].at[pl.ds(j * CHUNK, CHUNK)],
                    sem_g,
                )

        def out_descs(buf, r):
            return [
                pltpu.make_async_copy(
                    stage_v.at[buf].at[:, pl.ds(8 * s, 8), :],
                    out_hbm.at[s].at[pl.ds(r, K)],
                    sem_o,
                )
                for s in range(SUB)
            ]

        def transpose(buf):
            iota = jax.lax.iota(jnp.int32, NLANE)
            cols_mod = [(iota + e) % EMBED for e in range(EMBED)]
            rows_b = rows_v.at[buf]
            stage_b = stage_v.at[buf]

            @plsc.parallel_loop(0, G // NLANE, unroll=4)
            def t_body(t):
                rvec = iota + t * NLANE
                k = t // (CHUNK // NLANE)
                kvec = jnp.zeros((NLANE,), jnp.int32) + k
                jvec = rvec - k * CHUNK
                for e in range(EMBED):
                    v = plsc.load_gather(rows_b, [rvec, cols_mod[e]])
                    plsc.store_scatter(stage_b, [kvec, cols_mod[e], jvec], v)

        def step(g, cur):
            nxt = 1 - cur
            static = isinstance(g, int)
            r = row0 + g * K

            def when(cond_static, cond_traced, fn):
                if static:
                    if cond_static:
                        fn()
                else:
                    pl.when(cond_traced)(fn)

            when(g < LAST, g < LAST, lambda: pltpu.sync_copy(
                idx_hbm.at[pl.ds(r + K, K)],
                idx_v.at[pl.ds(nxt * K, K)],
            ))

            pltpu.make_async_copy(
                table_hbm.at[pl.ds(0, G)],
                rows_v.at[cur],
                sem_g,
            ).wait()

            when(g < LAST, g < LAST, lambda: fire(nxt, r + K))

            def retire():
                for d in out_descs(cur, row0):
                    d.wait()

            when(g >= 2, g >= 2, retire)

            transpose(cur)

            for d in out_descs(cur, r):
                d.start()

        pltpu.sync_copy(idx_hbm.at[pl.ds(row0, K)], idx_v.at[pl.ds(0, K)])
        fire(0, row0)
        step(0, 0)

        def pair(i, carry):
            g = 2 * i + 1
            step(g, 1)
            step(g + 1, 0)
            return carry

        lax.fori_loop(0, (N_GROUPS - 2) // 2, pair, 0)
        step(LAST, LAST % 2)

        for buf in ((LAST - 1) % 2, LAST % 2):
            for d in out_descs(buf, row0):
                d.wait()

    return body(idx2d, table)


def kernel(inputs, table):
    idx2d = inputs.astype(jnp.int32).reshape(N // CHUNK, CHUNK)
    out4d = _sc_gather(idx2d, table)
    return out4d.transpose(1, 3, 0, 2).reshape(N, EMBED)


# speedup vs baseline: 3.1394x; 1.2371x over previous
import functools

import jax
import jax.numpy as jnp
from jax import lax
from jax.experimental import pallas as pl
from jax.experimental.pallas import tpu as pltpu
from jax.experimental.pallas import tpu_sc as plsc

VOCAB = 100000
EMBED = 32
B = 16384
L = 50
N = B * L

NC = 2
NS = 16
NW = NC * NS
B_PER_W = N // NW
CHUNK = 128
K = 4
G = K * CHUNK
N_GROUPS = B_PER_W // G
ROWS_PER_W = B_PER_W // CHUNK
LAST = N_GROUPS - 1
NLANE = 16
SUB = EMBED // 8


def _sc_gather(idx2d, table):
    mesh = plsc.VectorSubcoreMesh(core_axis_name="c", subcore_axis_name="s")

    @functools.partial(
        pl.kernel,
        mesh=mesh,
        compiler_params=pltpu.CompilerParams(
            use_tc_tiling_on_sc=False, needs_layout_passes=False
        ),
        out_type=jax.ShapeDtypeStruct((SUB, N // CHUNK, 8, CHUNK), jnp.float32),
        scratch_types=[
            pltpu.VMEM((2 * K, CHUNK), jnp.int32),
            pltpu.VMEM((2, G, EMBED), jnp.float32),
            pltpu.VMEM((2, K, EMBED, CHUNK), jnp.float32),
            pltpu.SemaphoreType.DMA,
            pltpu.SemaphoreType.DMA,
        ],
    )
    def body(idx_hbm, table_hbm, out_hbm, idx_v, rows_v, stage_v, sem_g, sem_o):
        wid = lax.axis_index("s") * NC + lax.axis_index("c")
        row0 = wid * ROWS_PER_W

        def fire(buf, r):
            for j in range(K):
                pltpu.async_copy(
                    table_hbm.at[idx_v.at[buf * K + j]],
                    rows_v.at[buf].at[pl.ds(j * CHUNK, CHUNK)],
                    sem_g,
                )

        def out_descs(buf, r):
            return [
                pltpu.make_async_copy(
                    stage_v.at[buf].at[:, pl.ds(8 * s, 8), :],
                    out_hbm.at[s].at[pl.ds(r, K)],
                    sem_o,
                )
                for s in range(SUB)
            ]

        def transpose(buf):
            iota = jax.lax.iota(jnp.int32, NLANE)
            cols_mod = [(iota + e) % EMBED for e in range(EMBED)]
            rows_b = rows_v.at[buf]
            stage_b = stage_v.at[buf]

            @plsc.parallel_loop(0, G // NLANE, unroll=8)
            def t_body(t):
                rvec = iota + t * NLANE
                k = t // (CHUNK // NLANE)
                kvec = jnp.zeros((NLANE,), jnp.int32) + k
                jvec = rvec - k * CHUNK
                for e in range(EMBED):
                    v = plsc.load_gather(rows_b, [rvec, cols_mod[e]])
                    plsc.store_scatter(stage_b, [kvec, cols_mod[e], jvec], v)

        def step(g, cur):
            nxt = 1 - cur
            static = isinstance(g, int)
            r = row0 + g * K

            def when(cond_static, cond_traced, fn):
                if static:
                    if cond_static:
                        fn()
                else:
                    pl.when(cond_traced)(fn)

            when(g < LAST, g < LAST, lambda: pltpu.sync_copy(
                idx_hbm.at[pl.ds(r + K, K)],
                idx_v.at[pl.ds(nxt * K, K)],
            ))

            pltpu.make_async_copy(
                table_hbm.at[pl.ds(0, G)],
                rows_v.at[cur],
                sem_g,
            ).wait()

            when(g < LAST, g < LAST, lambda: fire(nxt, r + K))

            def retire():
                for d in out_descs(cur, row0):
                    d.wait()

            when(g >= 2, g >= 2, retire)

            transpose(cur)

            for d in out_descs(cur, r):
                d.start()

        pltpu.sync_copy(idx_hbm.at[pl.ds(row0, K)], idx_v.at[pl.ds(0, K)])
        fire(0, row0)
        step(0, 0)

        def pair(i, carry):
            g = 2 * i + 1
            step(g, 1)
            step(g + 1, 0)
            return carry

        lax.fori_loop(0, (N_GROUPS - 2) // 2, pair, 0)
        step(LAST, LAST % 2)

        for buf in ((LAST - 1) % 2, LAST % 2):
            for d in out_descs(buf, row0):
                d.wait()

    return body(idx2d, table)


def kernel(inputs, table):
    idx2d = inputs.astype(jnp.int32).reshape(N // CHUNK, CHUNK)
    out4d = _sc_gather(idx2d, table)
    return out4d.transpose(1, 3, 0, 2).reshape(N, EMBED)
